# baseline (speedup 1.0000x reference)
import functools

import jax
import jax.numpy as jnp
from jax import lax
from jax.experimental import pallas as pl
from jax.experimental.pallas import tpu as pltpu
from jax.experimental.pallas import tpu_sc as plsc

N = 10000
E = 320000
D = 128

NP = 10240
CHUNK = 96
NTILES = 32
EP = ((E + NTILES * CHUNK - 1) // (NTILES * CHUNK)) * (NTILES * CHUNK)
EDGES_PER_TILE = EP // NTILES
NCHUNKS = EDGES_PER_TILE // CHUNK
BLK = 1024
GRID = NP // BLK
NDEN = NP // 8

_F32 = jnp.float32
_HIGH = jax.lax.Precision.HIGHEST


def _dot(a, b):
    return lax.dot_general(a, b, (((1,), (0,)), ((), ())),
                           precision=_HIGH, preferred_element_type=_F32)


def _dot_fast(a, b):
    return lax.dot_general(a, b, (((1,), (0,)), ((), ())),
                           preferred_element_type=_F32)


def _leaky(x):
    return jnp.maximum(x, 0.2 * x)


def _tc_first_body(x_ref, w_ref, as_ref, ad_ref,
                   h_ref, aux_ref, ms_ref):
    i = pl.program_id(0)
    h = _dot_fast(x_ref[...], w_ref[...])
    h_ref[...] = h
    als = _dot(h, as_ref[...])
    ald = _dot(h, ad_ref[...])
    aux_ref[...] = jnp.concatenate(
        [als, ald, jnp.zeros((BLK, D - 32), _F32)], axis=1)
    bmax = jnp.max(als, axis=0, keepdims=True)
    bmax8 = jnp.broadcast_to(bmax, (8, 16))

    @pl.when(i == 0)
    def _():
        ms_ref[...] = bmax8

    @pl.when(i > 0)
    def _():
        ms_ref[...] = jnp.maximum(ms_ref[...], bmax8)


def _tc_first(xp, W, As16, Ad16):
    return pl.pallas_call(
        _tc_first_body,
        grid=(GRID,),
        in_specs=[
            pl.BlockSpec((BLK, D), lambda i: (i, 0)),
            pl.BlockSpec((D, D), lambda i: (0, 0)),
            pl.BlockSpec((D, 16), lambda i: (0, 0)),
            pl.BlockSpec((D, 16), lambda i: (0, 0)),
        ],
        out_specs=[
            pl.BlockSpec((BLK, D), lambda i: (i, 0)),
            pl.BlockSpec((BLK, D), lambda i: (i, 0)),
            pl.BlockSpec((8, 16), lambda i: (0, 0)),
        ],
        out_shape=[
            jax.ShapeDtypeStruct((NP, D), _F32),
            jax.ShapeDtypeStruct((NP, D), _F32),
            jax.ShapeDtypeStruct((8, 16), _F32),
        ],
    )(xp, W, As16, Ad16)


def _combine_block(accp, denp, h, aux, ms, p, b):
    als = aux[:, 0:16]
    ald = aux[:, 16:32]
    den_self = jnp.exp(_leaky(als + ald) - _leaky(ms[0:1, :] + ald))
    den4 = (denp[0] + denp[1] + den_self)[:, :4]
    ds4 = den_self[:, :4]
    den128 = _dot(den4, p)
    ds128 = _dot(ds4, p)
    num = accp[0] + accp[1] + ds128 * h
    return num / den128 + b


def _tc_combine_body(accp_ref, denp_ref, h_ref, aux_ref, ms_ref,
                     p_ref, b_ref, g_ref, be_ref, xres_ref,
                     w_ref, as_ref, ad_ref,
                     xn_ref, hn_ref, auxn_ref, msn_ref):
    i = pl.program_id(0)
    conv = _combine_block(accp_ref[...], denp_ref[...], h_ref[...],
                          aux_ref[...], ms_ref[...],
                          p_ref[...], b_ref[...])
    mu = jnp.mean(conv, axis=-1, keepdims=True)
    var = jnp.mean((conv - mu) ** 2, axis=-1, keepdims=True)
    ln = (conv - mu) / jnp.sqrt(var + 1e-5) * g_ref[...] + be_ref[...]
    act = jnp.where(ln > 0, ln, jnp.exp(ln) - 1.0)
    xn = act + xres_ref[...]
    xn_ref[...] = xn
    hn = _dot_fast(xn, w_ref[...])
    hn_ref[...] = hn
    alsn = _dot(hn, as_ref[...])
    aldn = _dot(hn, ad_ref[...])
    auxn_ref[...] = jnp.concatenate(
        [alsn, aldn, jnp.zeros((BLK, D - 32), _F32)], axis=1)
    bmax = jnp.max(alsn, axis=0, keepdims=True)
    bmax8 = jnp.broadcast_to(bmax, (8, 16))

    @pl.when(i == 0)
    def _():
        msn_ref[...] = bmax8

    @pl.when(i > 0)
    def _():
        msn_ref[...] = jnp.maximum(msn_ref[...], bmax8)


def _tc_combine(accp, denp, h, aux, msacc, P, b, g, be, xres, W, As16, Ad16):
    return pl.pallas_call(
        _tc_combine_body,
        grid=(GRID,),
        in_specs=[
            pl.BlockSpec((2, BLK, D), lambda i: (0, i, 0)),
            pl.BlockSpec((2, BLK, 16), lambda i: (0, i, 0)),
            pl.BlockSpec((BLK, D), lambda i: (i, 0)),
            pl.BlockSpec((BLK, D), lambda i: (i, 0)),
            pl.BlockSpec((8, 16), lambda i: (0, 0)),
            pl.BlockSpec((4, D), lambda i: (0, 0)),
            pl.BlockSpec((1, D), lambda i: (0, 0)),
            pl.BlockSpec((1, D), lambda i: (0, 0)),
            pl.BlockSpec((1, D), lambda i: (0, 0)),
            pl.BlockSpec((BLK, D), lambda i: (i, 0)),
            pl.BlockSpec((D, D), lambda i: (0, 0)),
            pl.BlockSpec((D, 16), lambda i: (0, 0)),
            pl.BlockSpec((D, 16), lambda i: (0, 0)),
        ],
        out_specs=[
            pl.BlockSpec((BLK, D), lambda i: (i, 0)),
            pl.BlockSpec((BLK, D), lambda i: (i, 0)),
            pl.BlockSpec((BLK, D), lambda i: (i, 0)),
            pl.BlockSpec((8, 16), lambda i: (0, 0)),
        ],
        out_shape=[
            jax.ShapeDtypeStruct((NP, D), _F32),
            jax.ShapeDtypeStruct((NP, D), _F32),
            jax.ShapeDtypeStruct((NP, D), _F32),
            jax.ShapeDtypeStruct((8, 16), _F32),
        ],
    )(accp, denp, h, aux, msacc, P, b, g, be, xres, W, As16, Ad16)


def _tc_final_body(accp_ref, denp_ref, h_ref, aux_ref, ms_ref,
                   p_ref, b_ref, x1_ref, x2_ref, out_ref):
    conv = _combine_block(accp_ref[...], denp_ref[...], h_ref[...],
                          aux_ref[...], ms_ref[...],
                          p_ref[...], b_ref[...])
    out_ref[:, 0:D] = x1_ref[...]
    out_ref[:, D:2 * D] = x2_ref[...]
    out_ref[:, 2 * D:3 * D] = conv


def _tc_final(accp, denp, h, aux, msacc, P, b, x1, x2):
    return pl.pallas_call(
        _tc_final_body,
        grid=(GRID,),
        in_specs=[
            pl.BlockSpec((2, BLK, D), lambda i: (0, i, 0)),
            pl.BlockSpec((2, BLK, 16), lambda i: (0, i, 0)),
            pl.BlockSpec((BLK, D), lambda i: (i, 0)),
            pl.BlockSpec((BLK, D), lambda i: (i, 0)),
            pl.BlockSpec((8, 16), lambda i: (0, 0)),
            pl.BlockSpec((4, D), lambda i: (0, 0)),
            pl.BlockSpec((1, D), lambda i: (0, 0)),
            pl.BlockSpec((BLK, D), lambda i: (i, 0)),
            pl.BlockSpec((BLK, D), lambda i: (i, 0)),
        ],
        out_specs=pl.BlockSpec((BLK, 3 * D), lambda i: (i, 0)),
        out_shape=jax.ShapeDtypeStruct((NP, 3 * D), _F32),
    )(accp, denp, h, aux, msacc, P, b, x1, x2)


def _sc_body(src_hbm, dst_hbm, dst8_hbm, dstm_hbm, oh_hbm, h_hbm, aux_hbm,
             ms_hbm, accp_hbm, denp_hbm,
             acc_sh, den_sh, src_v, dst_v, dst8_v, dstm_v,
             als_v, ex_v, rows_v,
             ms_v, zb128, sem0, sem1, sem2, sem3):
    cid = lax.axis_index("c")
    sid = lax.axis_index("s")

    zero16 = jnp.zeros((16,), _F32)
    for r in range(8):
        for j in range(8):
            zb128[r, pl.ds(j * 16, 16)] = zero16

    rows_per_tile = NP // 16
    dens_per_tile = NDEN // 16
    z0 = sid * rows_per_tile
    zd0 = sid * dens_per_tile

    def zbody(i, c):
        pltpu.sync_copy(zb128, acc_sh.at[pl.ds(z0 + i * 8, 8)])
        return c

    lax.fori_loop(0, rows_per_tile // 8, zbody, 0)

    def zdbody(i, c):
        pltpu.sync_copy(zb128, den_sh.at[pl.ds(zd0 + i * 8, 8)])
        return c

    lax.fori_loop(0, dens_per_tile // 8, zdbody, 0)
    pltpu.sync_copy(ms_hbm, ms_v)
    plsc.subcore_barrier()

    wid = sid * 2 + cid
    tbase = wid * EDGES_PER_TILE
    ms = ms_v[...]

    def chunk_body(k, c):
        base = tbase + k * CHUNK
        pltpu.sync_copy(src_hbm.at[pl.ds(base, CHUNK)], src_v)
        pltpu.sync_copy(dst_hbm.at[pl.ds(base, CHUNK)], dst_v)
        pltpu.sync_copy(dst8_hbm.at[pl.ds(base, CHUNK)], dst8_v)
        pltpu.sync_copy(dstm_hbm.at[pl.ds(base, CHUNK)], dstm_v)
        d0 = pltpu.async_copy(aux_hbm.at[src_v], als_v, sem0)
        d1 = pltpu.async_copy(aux_hbm.at[dst_v], rows_v, sem1)
        d0.wait()
        d1.wait()

        def exbody(e, c2):
            a_s = als_v[e, pl.ds(0, 16)]
            a_d = rows_v[e, pl.ds(16, 16)]
            s = a_s + a_d
            el = jnp.maximum(s, 0.2 * s)
            mm = ms + a_d
            ml = jnp.maximum(mm, 0.2 * mm)
            ex_v[e, :] = jnp.exp(el - ml)
            return c2

        lax.fori_loop(0, CHUNK, exbody, 0)
        d2 = pltpu.async_copy(oh_hbm.at[dstm_v], als_v, sem0)
        d3 = pltpu.async_copy(h_hbm.at[src_v], rows_v, sem1)
        d2.wait()
        d3.wait()

        def sbody(e, c2):
            exv = ex_v[e, :]
            for j in range(8):
                sl = pl.ds(j * 16, 16)
                als_v[e, sl] = als_v[e, sl] * exv
            for hh in range(4):
                a = exv[hh]
                for j in range(2):
                    sl = pl.ds(hh * 32 + j * 16, 16)
                    rows_v[e, sl] = rows_v[e, sl] * a
            return c2

        lax.fori_loop(0, CHUNK, sbody, 0)
        d4 = pltpu.async_copy(als_v, den_sh.at[dst8_v], sem2, add=True)
        d5 = pltpu.async_copy(rows_v, acc_sh.at[dst_v], sem3, add=True)
        d4.wait()
        d5.wait()
        return c

    lax.fori_loop(0, NCHUNKS, chunk_body, 0)
    plsc.subcore_barrier()

    pltpu.sync_copy(acc_sh.at[pl.ds(z0, rows_per_tile)],
                    accp_hbm.at[cid, pl.ds(z0, rows_per_tile)])
    pltpu.sync_copy(den_sh.at[pl.ds(zd0, dens_per_tile)],
                    denp_hbm.at[cid, pl.ds(zd0, dens_per_tile)])


def _sc_edge(*args):
    return _make_sc_edge()(*args)


@functools.cache
def _make_sc_edge():
    return pl.kernel(
        _sc_body,
        out_type=(
            jax.ShapeDtypeStruct((2, NP, D), _F32),
            jax.ShapeDtypeStruct((2, NDEN, D), _F32),
        ),
        mesh=plsc.VectorSubcoreMesh(core_axis_name="c", subcore_axis_name="s"),
        scratch_types=(
            pltpu.VMEM_SHARED((NP, D), _F32),
            pltpu.VMEM_SHARED((NDEN, D), _F32),
            pltpu.VMEM((CHUNK,), jnp.int32),
            pltpu.VMEM((CHUNK,), jnp.int32),
            pltpu.VMEM((CHUNK,), jnp.int32),
            pltpu.VMEM((CHUNK,), jnp.int32),
            pltpu.VMEM((CHUNK, D), _F32),
            pltpu.VMEM((CHUNK, 16), _F32),
            pltpu.VMEM((CHUNK, D), _F32),
            pltpu.VMEM((16,), _F32),
            pltpu.VMEM((8, D), _F32),
            pltpu.SemaphoreType.DMA,
            pltpu.SemaphoreType.DMA,
            pltpu.SemaphoreType.DMA,
            pltpu.SemaphoreType.DMA,
        ),
    )


def _build_as16(a):
    H, hc = a.shape
    eye = jnp.eye(H, dtype=_F32)
    As = (a[:, :, None] * eye[:, None, :]).reshape(H * hc, H)
    return jnp.tile(As, (1, 16 // H))


def kernel(x, edge_index, W0, a_src0, a_dst0, b0, g0, be0,
           W1, a_src1, a_dst1, b1, g1, be1, W2, a_src2, a_dst2, b2):
    xp = jnp.zeros((NP, D), _F32).at[:N].set(x)
    pad = jnp.full((EP - E,), N, jnp.int32)
    srcp = jnp.concatenate([edge_index[0].astype(jnp.int32), pad])
    dstp = jnp.concatenate([edge_index[1].astype(jnp.int32), pad])
    dst8p = dstp // 8
    dstmp = dstp % 8
    onehot = jnp.repeat(jnp.eye(8, dtype=_F32), 16, axis=1)

    P = jnp.repeat(jnp.eye(4, dtype=_F32), 32, axis=1)
    b0r, g0r, be0r = b0.reshape(1, D), g0.reshape(1, D), be0.reshape(1, D)
    b1r, g1r, be1r = b1.reshape(1, D), g1.reshape(1, D), be1.reshape(1, D)
    b2r = b2.reshape(1, D)
    As0, Ad0 = _build_as16(a_src0), _build_as16(a_dst0)
    As1, Ad1 = _build_as16(a_src1), _build_as16(a_dst1)
    As2, Ad2 = _build_as16(a_src2), _build_as16(a_dst2)

    h0, aux0, ms0 = _tc_first(xp, W0, As0, Ad0)
    acc0, den0 = _sc_edge(srcp, dstp, dst8p, dstmp, onehot, h0, aux0, ms0[0])
    x1, h1, aux1, ms1 = _tc_combine(acc0, den0.reshape(2, NP, 16), h0, aux0,
                                    ms0, P, b0r, g0r, be0r, xp, W1, As1, Ad1)
    acc1, den1 = _sc_edge(srcp, dstp, dst8p, dstmp, onehot, h1, aux1, ms1[0])
    x2, h2, aux2, ms2 = _tc_combine(acc1, den1.reshape(2, NP, 16), h1, aux1,
                                    ms1, P, b1r, g1r, be1r, x1, W2, As2, Ad2)
    acc2, den2 = _sc_edge(srcp, dstp, dst8p, dstmp, onehot, h2, aux2, ms2[0])
    out = _tc_final(acc2, den2.reshape(2, NP, 16), h2, aux2, ms2, P, b2r, x1, x2)
    return out[:N]


# speedup vs baseline: 1.1439x; 1.1439x over previous
import functools

import jax
import jax.numpy as jnp
from jax import lax
from jax.experimental import pallas as pl
from jax.experimental.pallas import tpu as pltpu
from jax.experimental.pallas import tpu_sc as plsc

N = 10000
E = 320000
D = 128

NP = 10240
CHUNK = 56
NTILES = 32
EP = ((E + NTILES * CHUNK - 1) // (NTILES * CHUNK)) * (NTILES * CHUNK)
EDGES_PER_TILE = EP // NTILES
NCHUNKS = EDGES_PER_TILE // CHUNK
BLK = 1024
GRID = NP // BLK
NDEN = NP // 8

_F32 = jnp.float32
_HIGH = jax.lax.Precision.HIGHEST


def _dot(a, b):
    return lax.dot_general(a, b, (((1,), (0,)), ((), ())),
                           precision=_HIGH, preferred_element_type=_F32)


def _dot_fast(a, b):
    return lax.dot_general(a, b, (((1,), (0,)), ((), ())),
                           preferred_element_type=_F32)


def _leaky(x):
    return jnp.maximum(x, 0.2 * x)


def _tc_first_body(x_ref, w_ref, as_ref, ad_ref,
                   h_ref, aux_ref, ms_ref):
    i = pl.program_id(0)
    h = _dot_fast(x_ref[...], w_ref[...])
    h_ref[...] = h
    als = _dot(h, as_ref[...])
    ald = _dot(h, ad_ref[...])
    aux_ref[...] = jnp.concatenate(
        [als, ald, jnp.zeros((BLK, D - 32), _F32)], axis=1)
    bmax = jnp.max(als, axis=0, keepdims=True)
    bmax8 = jnp.broadcast_to(bmax, (8, 16))

    @pl.when(i == 0)
    def _():
        ms_ref[...] = bmax8

    @pl.when(i > 0)
    def _():
        ms_ref[...] = jnp.maximum(ms_ref[...], bmax8)


def _tc_first(xp, W, As16, Ad16):
    return pl.pallas_call(
        _tc_first_body,
        grid=(GRID,),
        in_specs=[
            pl.BlockSpec((BLK, D), lambda i: (i, 0)),
            pl.BlockSpec((D, D), lambda i: (0, 0)),
            pl.BlockSpec((D, 16), lambda i: (0, 0)),
            pl.BlockSpec((D, 16), lambda i: (0, 0)),
        ],
        out_specs=[
            pl.BlockSpec((BLK, D), lambda i: (i, 0)),
            pl.BlockSpec((BLK, D), lambda i: (i, 0)),
            pl.BlockSpec((8, 16), lambda i: (0, 0)),
        ],
        out_shape=[
            jax.ShapeDtypeStruct((NP, D), _F32),
            jax.ShapeDtypeStruct((NP, D), _F32),
            jax.ShapeDtypeStruct((8, 16), _F32),
        ],
    )(xp, W, As16, Ad16)


def _combine_block(accp, denp, h, aux, ms, p, b):
    als = aux[:, 0:16]
    ald = aux[:, 16:32]
    den_self = jnp.exp(_leaky(als + ald) - _leaky(ms[0:1, :] + ald))
    den4 = (denp[0] + denp[1] + den_self)[:, :4]
    ds4 = den_self[:, :4]
    den128 = _dot(den4, p)
    ds128 = _dot(ds4, p)
    num = accp[0] + accp[1] + ds128 * h
    return num / den128 + b


def _tc_combine_body(accp_ref, denp_ref, h_ref, aux_ref, ms_ref,
                     p_ref, b_ref, g_ref, be_ref, xres_ref,
                     w_ref, as_ref, ad_ref,
                     xn_ref, hn_ref, auxn_ref, msn_ref):
    i = pl.program_id(0)
    conv = _combine_block(accp_ref[...], denp_ref[...], h_ref[...],
                          aux_ref[...], ms_ref[...],
                          p_ref[...], b_ref[...])
    mu = jnp.mean(conv, axis=-1, keepdims=True)
    var = jnp.mean((conv - mu) ** 2, axis=-1, keepdims=True)
    ln = (conv - mu) / jnp.sqrt(var + 1e-5) * g_ref[...] + be_ref[...]
    act = jnp.where(ln > 0, ln, jnp.exp(ln) - 1.0)
    xn = act + xres_ref[...]
    xn_ref[...] = xn
    hn = _dot_fast(xn, w_ref[...])
    hn_ref[...] = hn
    alsn = _dot(hn, as_ref[...])
    aldn = _dot(hn, ad_ref[...])
    auxn_ref[...] = jnp.concatenate(
        [alsn, aldn, jnp.zeros((BLK, D - 32), _F32)], axis=1)
    bmax = jnp.max(alsn, axis=0, keepdims=True)
    bmax8 = jnp.broadcast_to(bmax, (8, 16))

    @pl.when(i == 0)
    def _():
        msn_ref[...] = bmax8

    @pl.when(i > 0)
    def _():
        msn_ref[...] = jnp.maximum(msn_ref[...], bmax8)


def _tc_combine(accp, denp, h, aux, msacc, P, b, g, be, xres, W, As16, Ad16):
    return pl.pallas_call(
        _tc_combine_body,
        grid=(GRID,),
        in_specs=[
            pl.BlockSpec((2, BLK, D), lambda i: (0, i, 0)),
            pl.BlockSpec((2, BLK, 16), lambda i: (0, i, 0)),
            pl.BlockSpec((BLK, D), lambda i: (i, 0)),
            pl.BlockSpec((BLK, D), lambda i: (i, 0)),
            pl.BlockSpec((8, 16), lambda i: (0, 0)),
            pl.BlockSpec((4, D), lambda i: (0, 0)),
            pl.BlockSpec((1, D), lambda i: (0, 0)),
            pl.BlockSpec((1, D), lambda i: (0, 0)),
            pl.BlockSpec((1, D), lambda i: (0, 0)),
            pl.BlockSpec((BLK, D), lambda i: (i, 0)),
            pl.BlockSpec((D, D), lambda i: (0, 0)),
            pl.BlockSpec((D, 16), lambda i: (0, 0)),
            pl.BlockSpec((D, 16), lambda i: (0, 0)),
        ],
        out_specs=[
            pl.BlockSpec((BLK, D), lambda i: (i, 0)),
            pl.BlockSpec((BLK, D), lambda i: (i, 0)),
            pl.BlockSpec((BLK, D), lambda i: (i, 0)),
            pl.BlockSpec((8, 16), lambda i: (0, 0)),
        ],
        out_shape=[
            jax.ShapeDtypeStruct((NP, D), _F32),
            jax.ShapeDtypeStruct((NP, D), _F32),
            jax.ShapeDtypeStruct((NP, D), _F32),
            jax.ShapeDtypeStruct((8, 16), _F32),
        ],
    )(accp, denp, h, aux, msacc, P, b, g, be, xres, W, As16, Ad16)


def _tc_final_body(accp_ref, denp_ref, h_ref, aux_ref, ms_ref,
                   p_ref, b_ref, x1_ref, x2_ref, out_ref):
    conv = _combine_block(accp_ref[...], denp_ref[...], h_ref[...],
                          aux_ref[...], ms_ref[...],
                          p_ref[...], b_ref[...])
    out_ref[:, 0:D] = x1_ref[...]
    out_ref[:, D:2 * D] = x2_ref[...]
    out_ref[:, 2 * D:3 * D] = conv


def _tc_final(accp, denp, h, aux, msacc, P, b, x1, x2):
    return pl.pallas_call(
        _tc_final_body,
        grid=(GRID,),
        in_specs=[
            pl.BlockSpec((2, BLK, D), lambda i: (0, i, 0)),
            pl.BlockSpec((2, BLK, 16), lambda i: (0, i, 0)),
            pl.BlockSpec((BLK, D), lambda i: (i, 0)),
            pl.BlockSpec((BLK, D), lambda i: (i, 0)),
            pl.BlockSpec((8, 16), lambda i: (0, 0)),
            pl.BlockSpec((4, D), lambda i: (0, 0)),
            pl.BlockSpec((1, D), lambda i: (0, 0)),
            pl.BlockSpec((BLK, D), lambda i: (i, 0)),
            pl.BlockSpec((BLK, D), lambda i: (i, 0)),
        ],
        out_specs=pl.BlockSpec((BLK, 3 * D), lambda i: (i, 0)),
        out_shape=jax.ShapeDtypeStruct((NP, 3 * D), _F32),
    )(accp, denp, h, aux, msacc, P, b, x1, x2)


def _sc_body(eidx_hbm, oh_hbm, h_hbm, aux_hbm,
             ms_hbm, accp_hbm, denp_hbm,
             acc_sh, den_sh, eidx_v,
             als_v, ald_v, oh_v, rows_v, ex_v,
             ms_v, zb128, sem0, sem1, sem2, sem3):
    cid = lax.axis_index("c")
    sid = lax.axis_index("s")

    zero16 = jnp.zeros((16,), _F32)
    for r in range(8):
        for j in range(8):
            zb128[r, pl.ds(j * 16, 16)] = zero16

    rows_per_tile = NP // 16
    dens_per_tile = NDEN // 16
    z0 = sid * rows_per_tile
    zd0 = sid * dens_per_tile

    def zbody(i, c):
        pltpu.sync_copy(zb128, acc_sh.at[pl.ds(z0 + i * 8, 8)])
        return c

    lax.fori_loop(0, rows_per_tile // 8, zbody, 0)

    def zdbody(i, c):
        pltpu.sync_copy(zb128, den_sh.at[pl.ds(zd0 + i * 8, 8)])
        return c

    lax.fori_loop(0, dens_per_tile // 8, zdbody, 0)
    pltpu.sync_copy(ms_hbm, ms_v)
    plsc.subcore_barrier()

    wid = sid * 2 + cid
    tbase = wid * EDGES_PER_TILE
    ms = ms_v[...]

    def chunk_body(k, c):
        pltpu.sync_copy(eidx_hbm.at[wid * NCHUNKS + k], eidx_v)
        src_i = eidx_v.at[0]
        dst_i = eidx_v.at[1]
        dst8_i = eidx_v.at[2]
        dstm_i = eidx_v.at[3]
        d0 = pltpu.async_copy(aux_hbm.at[src_i], als_v, sem0)
        d1 = pltpu.async_copy(aux_hbm.at[dst_i], ald_v, sem1)
        d2 = pltpu.async_copy(oh_hbm.at[dstm_i], oh_v, sem2)
        d3 = pltpu.async_copy(h_hbm.at[src_i], rows_v, sem3)
        d0.wait()
        d1.wait()
        d2.wait()

        def exbody(e, c2):
            a_s = als_v[e, pl.ds(0, 16)]
            a_d = ald_v[e, pl.ds(16, 16)]
            s = a_s + a_d
            el = jnp.maximum(s, 0.2 * s)
            mm = ms + a_d
            ml = jnp.maximum(mm, 0.2 * mm)
            exv = jnp.exp(el - ml)
            ex_v[e, :] = exv
            for j in range(8):
                sl = pl.ds(j * 16, 16)
                oh_v[e, sl] = oh_v[e, sl] * exv
            return c2

        lax.fori_loop(0, CHUNK, exbody, 0)
        d4 = pltpu.async_copy(oh_v, den_sh.at[dst8_i], sem2, add=True)
        d3.wait()

        def sbody(e, c2):
            exv = ex_v[e, :]
            for hh in range(4):
                a = exv[hh]
                for j in range(2):
                    sl = pl.ds(hh * 32 + j * 16, 16)
                    rows_v[e, sl] = rows_v[e, sl] * a
            return c2

        lax.fori_loop(0, CHUNK, sbody, 0)
        d5 = pltpu.async_copy(rows_v, acc_sh.at[dst_i], sem3, add=True)
        d4.wait()
        d5.wait()
        return c

    lax.fori_loop(0, NCHUNKS, chunk_body, 0)
    plsc.subcore_barrier()

    pltpu.sync_copy(acc_sh.at[pl.ds(z0, rows_per_tile)],
                    accp_hbm.at[cid, pl.ds(z0, rows_per_tile)])
    pltpu.sync_copy(den_sh.at[pl.ds(zd0, dens_per_tile)],
                    denp_hbm.at[cid, pl.ds(zd0, dens_per_tile)])


def _sc_edge(*args):
    return _make_sc_edge()(*args)


@functools.cache
def _make_sc_edge():
    return pl.kernel(
        _sc_body,
        out_type=(
            jax.ShapeDtypeStruct((2, NP, D), _F32),
            jax.ShapeDtypeStruct((2, NDEN, D), _F32),
        ),
        mesh=plsc.VectorSubcoreMesh(core_axis_name="c", subcore_axis_name="s"),
        scratch_types=(
            pltpu.VMEM_SHARED((NP, D), _F32),
            pltpu.VMEM_SHARED((NDEN, D), _F32),
            pltpu.VMEM((4, CHUNK), jnp.int32),
            pltpu.VMEM((CHUNK, D), _F32),
            pltpu.VMEM((CHUNK, D), _F32),
            pltpu.VMEM((CHUNK, D), _F32),
            pltpu.VMEM((CHUNK, D), _F32),
            pltpu.VMEM((CHUNK, 16), _F32),
            pltpu.VMEM((16,), _F32),
            pltpu.VMEM((8, D), _F32),
            pltpu.SemaphoreType.DMA,
            pltpu.SemaphoreType.DMA,
            pltpu.SemaphoreType.DMA,
            pltpu.SemaphoreType.DMA,
        ),
    )


def _build_as16(a):
    H, hc = a.shape
    eye = jnp.eye(H, dtype=_F32)
    As = (a[:, :, None] * eye[:, None, :]).reshape(H * hc, H)
    return jnp.tile(As, (1, 16 // H))


def kernel(x, edge_index, W0, a_src0, a_dst0, b0, g0, be0,
           W1, a_src1, a_dst1, b1, g1, be1, W2, a_src2, a_dst2, b2):
    xp = jnp.zeros((NP, D), _F32).at[:N].set(x)
    pad = jnp.full((EP - E,), N, jnp.int32)
    srcp = jnp.concatenate([edge_index[0].astype(jnp.int32), pad])
    dstp = jnp.concatenate([edge_index[1].astype(jnp.int32), pad])
    dst8p = dstp // 8
    dstmp = dstp % 8
    eidx = jnp.stack([srcp, dstp, dst8p, dstmp])
    eidx = eidx.reshape(4, EP // CHUNK, CHUNK).transpose(1, 0, 2)
    onehot = jnp.repeat(jnp.eye(8, dtype=_F32), 16, axis=1)

    P = jnp.repeat(jnp.eye(4, dtype=_F32), 32, axis=1)
    b0r, g0r, be0r = b0.reshape(1, D), g0.reshape(1, D), be0.reshape(1, D)
    b1r, g1r, be1r = b1.reshape(1, D), g1.reshape(1, D), be1.reshape(1, D)
    b2r = b2.reshape(1, D)
    As0, Ad0 = _build_as16(a_src0), _build_as16(a_dst0)
    As1, Ad1 = _build_as16(a_src1), _build_as16(a_dst1)
    As2, Ad2 = _build_as16(a_src2), _build_as16(a_dst2)

    h0, aux0, ms0 = _tc_first(xp, W0, As0, Ad0)
    acc0, den0 = _sc_edge(eidx, onehot, h0, aux0, ms0[0])
    x1, h1, aux1, ms1 = _tc_combine(acc0, den0.reshape(2, NP, 16), h0, aux0,
                                    ms0, P, b0r, g0r, be0r, xp, W1, As1, Ad1)
    acc1, den1 = _sc_edge(eidx, onehot, h1, aux1, ms1[0])
    x2, h2, aux2, ms2 = _tc_combine(acc1, den1.reshape(2, NP, 16), h1, aux1,
                                    ms1, P, b1r, g1r, be1r, x1, W2, As2, Ad2)
    acc2, den2 = _sc_edge(eidx, onehot, h2, aux2, ms2[0])
    out = _tc_final(acc2, den2.reshape(2, NP, 16), h2, aux2, ms2, P, b2r, x1, x2)
    return out[:N]
